# baseline (speedup 1.0000x reference)
import functools

import jax
import jax.numpy as jnp
from jax import lax
from jax.experimental import pallas as pl
from jax.experimental.pallas import tpu as pltpu
from jax.experimental.pallas import tpu_sc as plsc

_V = 1000
_VP = 1008
_D = 64
_L = 16
_CH = 16
_NCH = 63
_BIG = 1 << 30


def _row_argmax(buf, rr):
    lane = lax.iota(jnp.int32, _L)
    neg = jnp.full((_L,), float("-inf"), jnp.float32)
    bv = [neg, neg, neg, neg]
    bi = [jnp.zeros((_L,), jnp.int32)] * 4
    for c in range(_NCH):
        v = buf[rr, pl.ds(c * _L, _L)]
        if c == _NCH - 1:
            v = jnp.where(lane < (_V - (_NCH - 1) * _L), v, neg)
        k = c & 3
        gt = v > bv[k]
        bv[k] = jnp.maximum(bv[k], v)
        bi[k] = jnp.where(gt, jnp.full((_L,), c * _L, jnp.int32), bi[k])

    def merge(av, ai, bv_, bi_):
        gt = av > bv_
        eq = av == bv_
        lt = ai < bi_
        ta = jnp.logical_or(gt, jnp.logical_and(eq, lt))
        return jnp.where(ta, av, bv_), jnp.where(ta, ai, bi_)

    v01, i01 = merge(bv[0], bi[0], bv[1], bi[1])
    v23, i23 = merge(bv[2], bi[2], bv[3], bi[3])
    v, i = merge(v01, i01, v23, i23)
    idx = i + lane
    m = jnp.max(v)
    eqm = v == jnp.full((_L,), m, jnp.float32)
    cand = jnp.where(eqm, idx, jnp.full((_L,), _BIG, jnp.int32))
    return jnp.min(cand)


def kernel(x, W):
    B, N, V = x.shape
    R = B * N
    info = plsc.get_sparse_core_info()
    NC, NS = info.num_cores, info.num_subcores
    NW = NC * NS
    rpw = R // NW
    n_chunks = rpw // _CH
    x2 = x.reshape(R, V)
    wf = W.reshape(-1)
    mesh = plsc.VectorSubcoreMesh(core_axis_name="c", subcore_axis_name="s")

    @functools.partial(
        pl.kernel,
        out_type=jax.ShapeDtypeStruct((R * _D,), jnp.float32),
        mesh=mesh,
        scratch_types=[
            pltpu.VMEM((_CH, _VP), jnp.float32),
            pltpu.VMEM((_CH, _VP), jnp.float32),
            pltpu.VMEM((_V * _D,), jnp.float32),
            pltpu.VMEM((_CH * _D,), jnp.float32),
            pltpu.VMEM((_CH * _D,), jnp.float32),
            pltpu.SemaphoreType.DMA,
            pltpu.SemaphoreType.DMA,
            pltpu.SemaphoreType.DMA,
            pltpu.SemaphoreType.DMA,
            pltpu.SemaphoreType.DMA,
        ],
        compiler_params=pltpu.CompilerParams(
            use_tc_tiling_on_sc=False, needs_layout_passes=False),
    )
    def run(x_hbm, w_hbm, o_hbm, xb0, xb1, wtab, rb0, rb1,
            is0, is1, ws, os0, os1):
        cid = lax.axis_index("c")
        sid = lax.axis_index("s")
        wid = sid * NC + cid
        row0 = wid * rpw

        pltpu.async_copy(w_hbm, wtab, ws).wait()

        def in_src(g):
            return x_hbm.at[pl.ds(row0 + g * _CH, _CH), :]

        def in_dst(buf):
            return buf.at[:, pl.ds(0, _V)]

        def out_dst(g):
            return o_hbm.at[pl.ds((row0 + g * _CH) * _D, _CH * _D)]

        xbufs = (xb0, xb1)
        rbufs = (rb0, rb1)
        isems = (is0, is1)
        osems = (os0, os1)

        pltpu.async_copy(in_src(0), in_dst(xb0), is0)
        pltpu.async_copy(in_src(1), in_dst(xb1), is1)

        lane = lax.iota(jnp.int32, _L)

        def do_chunk(g, b):
            buf, rb = xbufs[b], rbufs[b]
            pltpu.make_async_copy(in_src(g), in_dst(buf), isems[b]).wait()

            @pl.when(g >= 2)
            def _():
                pltpu.make_async_copy(rb, out_dst(g - 2), osems[b]).wait()

            def row_body(rr, carry):
                tok = _row_argmax(buf, rr)
                i0 = jnp.full((_L,), tok * _D, jnp.int32) + lane
                for k2 in range(_D // _L):
                    rb[pl.ds(rr * _D + k2 * _L, _L)] = plsc.load_gather(
                        wtab, [i0 + (k2 * _L)])
                return carry
            lax.fori_loop(0, _CH, row_body, 0)

            @pl.when(g + 2 < n_chunks)
            def _():
                pltpu.async_copy(in_src(g + 2), in_dst(buf), isems[b])

            pltpu.async_copy(rb, out_dst(g), osems[b])

        def outer(gp, carry):
            do_chunk(2 * gp, 0)
            do_chunk(2 * gp + 1, 1)
            return carry
        lax.fori_loop(0, n_chunks // 2, outer, 0)

        pltpu.make_async_copy(rb0, out_dst(n_chunks - 2), osems[0]).wait()
        pltpu.make_async_copy(rb1, out_dst(n_chunks - 1), osems[1]).wait()

    out = run(x2, wf)
    return out.reshape(B, N, _D)


# speedup vs baseline: 1.2750x; 1.2750x over previous
import functools

import jax
import jax.numpy as jnp
from jax import lax
from jax.experimental import pallas as pl
from jax.experimental.pallas import tpu as pltpu
from jax.experimental.pallas import tpu_sc as plsc

_V = 1000
_D = 64
_L = 16
_CH = 32
_BR = 512
_BIG = 1 << 30


def _argmax_block(x_ref, tok_ref):
    xb = x_ref[...]
    m = jnp.max(xb, axis=1, keepdims=True)
    idx = lax.broadcasted_iota(jnp.int32, xb.shape, 1)
    cand = jnp.where(xb == m, idx, _BIG)
    tok_ref[...] = jnp.min(cand, axis=1, keepdims=True)


def _tc_argmax(x2):
    R, V = x2.shape
    return pl.pallas_call(
        _argmax_block,
        grid=(R // _BR,),
        in_specs=[pl.BlockSpec((_BR, V), lambda i: (i, 0))],
        out_specs=pl.BlockSpec((_BR, 1), lambda i: (i, 0)),
        out_shape=jax.ShapeDtypeStruct((R, 1), jnp.int32),
        compiler_params=pltpu.CompilerParams(
            dimension_semantics=("arbitrary",)),
    )(x2)


def kernel(x, W):
    B, N, V = x.shape
    R = B * N
    x2 = x.reshape(R, V)
    tokens = _tc_argmax(x2).reshape(R)

    info = plsc.get_sparse_core_info()
    NC, NS = info.num_cores, info.num_subcores
    NW = NC * NS
    rpw = R // NW
    n_chunks = rpw // _CH
    wf = W.reshape(-1)
    mesh = plsc.VectorSubcoreMesh(core_axis_name="c", subcore_axis_name="s")

    @functools.partial(
        pl.kernel,
        out_type=jax.ShapeDtypeStruct((R * _D,), jnp.float32),
        mesh=mesh,
        scratch_types=[
            pltpu.VMEM((rpw,), jnp.int32),
            pltpu.VMEM((_V * _D,), jnp.float32),
            pltpu.VMEM((_CH * _D,), jnp.float32),
            pltpu.VMEM((_CH * _D,), jnp.float32),
            pltpu.SemaphoreType.DMA,
            pltpu.SemaphoreType.DMA,
            pltpu.SemaphoreType.DMA,
            pltpu.SemaphoreType.DMA,
        ],
        compiler_params=pltpu.CompilerParams(
            use_tc_tiling_on_sc=False, needs_layout_passes=False),
    )
    def run(t_hbm, w_hbm, o_hbm, tokbuf, wtab, rb0, rb1, ts, ws, os0, os1):
        cid = lax.axis_index("c")
        sid = lax.axis_index("s")
        wid = sid * NC + cid
        row0 = wid * rpw

        pltpu.async_copy(w_hbm, wtab, ws)
        pltpu.async_copy(t_hbm.at[pl.ds(row0, rpw)], tokbuf, ts)
        pltpu.make_async_copy(w_hbm, wtab, ws).wait()
        pltpu.make_async_copy(t_hbm.at[pl.ds(row0, rpw)], tokbuf, ts).wait()

        rbufs = (rb0, rb1)
        osems = (os0, os1)

        def out_dst(g):
            return o_hbm.at[pl.ds((row0 + g * _CH) * _D, _CH * _D)]

        def do_chunk(g, b):
            rb = rbufs[b]

            @pl.when(g >= 2)
            def _():
                pltpu.make_async_copy(rb, out_dst(g - 2), osems[b]).wait()

            def group_body(h, carry):
                tv = tokbuf[pl.ds(g * _CH + h * _L, _L)] * _D
                r0 = h * _L
                for rr in range(_L):
                    base = tv[rr]
                    for k in range(_D // _L):
                        rb[pl.ds((r0 + rr) * _D + k * _L, _L)] = (
                            wtab[pl.ds(base + k * _L, _L)])
                return carry
            lax.fori_loop(0, _CH // _L, group_body, 0)

            pltpu.async_copy(rb, out_dst(g), osems[b])

        def outer(gp, carry):
            do_chunk(2 * gp, 0)
            do_chunk(2 * gp + 1, 1)
            return carry
        lax.fori_loop(0, n_chunks // 2, outer, 0)

        pltpu.make_async_copy(rb0, out_dst(n_chunks - 2), osems[0]).wait()
        pltpu.make_async_copy(rb1, out_dst(n_chunks - 1), osems[1]).wait()

    out = run(tokens, wf)
    return out.reshape(B, N, _D)


# speedup vs baseline: 1.8068x; 1.4171x over previous
import functools

import jax
import jax.numpy as jnp
from jax import lax
from jax.experimental import pallas as pl
from jax.experimental.pallas import tpu as pltpu
from jax.experimental.pallas import tpu_sc as plsc

_V = 1000
_D = 64
_L = 16
_CH = 32
_BB = 64
_BIG = 1 << 30


def _argmax_block(x_ref, tok_ref):
    xb = x_ref[...]
    m = jnp.max(xb, axis=2, keepdims=True)
    idx = lax.broadcasted_iota(jnp.int32, xb.shape, 2)
    cand = jnp.where(xb == m, idx, _BIG)
    tok_ref[...] = jnp.min(cand, axis=2)


def _tc_argmax(x):
    B, N, V = x.shape
    return pl.pallas_call(
        _argmax_block,
        grid=(B // _BB,),
        in_specs=[pl.BlockSpec((_BB, N, V), lambda i: (i, 0, 0))],
        out_specs=pl.BlockSpec((_BB, N), lambda i: (i, 0)),
        out_shape=jax.ShapeDtypeStruct((B, N), jnp.int32),
        compiler_params=pltpu.CompilerParams(
            dimension_semantics=("arbitrary",)),
    )(x)


def kernel(x, W):
    B, N, V = x.shape
    R = B * N
    tokens = _tc_argmax(x).reshape(R)

    info = plsc.get_sparse_core_info()
    NC, NS = info.num_cores, info.num_subcores
    NW = NC * NS
    rpw = R // NW
    n_chunks = rpw // _CH
    wf = W.reshape(-1)
    mesh = plsc.VectorSubcoreMesh(core_axis_name="c", subcore_axis_name="s")

    @functools.partial(
        pl.kernel,
        out_type=jax.ShapeDtypeStruct((R * _D,), jnp.float32),
        mesh=mesh,
        scratch_types=[
            pltpu.VMEM((rpw,), jnp.int32),
            pltpu.VMEM((_V * _D,), jnp.float32),
            pltpu.VMEM((_CH * _D,), jnp.float32),
            pltpu.VMEM((_CH * _D,), jnp.float32),
            pltpu.SemaphoreType.DMA,
            pltpu.SemaphoreType.DMA,
            pltpu.SemaphoreType.DMA,
            pltpu.SemaphoreType.DMA,
        ],
        compiler_params=pltpu.CompilerParams(
            use_tc_tiling_on_sc=False, needs_layout_passes=False),
    )
    def run(t_hbm, w_hbm, o_hbm, tokbuf, wtab, rb0, rb1, ts, ws, os0, os1):
        cid = lax.axis_index("c")
        sid = lax.axis_index("s")
        wid = sid * NC + cid
        row0 = wid * rpw

        pltpu.async_copy(w_hbm, wtab, ws)
        pltpu.async_copy(t_hbm.at[pl.ds(row0, rpw)], tokbuf, ts)
        pltpu.make_async_copy(w_hbm, wtab, ws).wait()
        pltpu.make_async_copy(t_hbm.at[pl.ds(row0, rpw)], tokbuf, ts).wait()

        rbufs = (rb0, rb1)
        osems = (os0, os1)

        def out_dst(g):
            return o_hbm.at[pl.ds((row0 + g * _CH) * _D, _CH * _D)]

        def do_chunk(g, b):
            rb = rbufs[b]

            @pl.when(g >= 2)
            def _():
                pltpu.make_async_copy(rb, out_dst(g - 2), osems[b]).wait()

            def group_body(h, carry):
                tv = tokbuf[pl.ds(g * _CH + h * _L, _L)] * _D
                r0 = h * _L
                for rr in range(_L):
                    base = tv[rr]
                    for k in range(_D // _L):
                        rb[pl.ds((r0 + rr) * _D + k * _L, _L)] = (
                            wtab[pl.ds(base + k * _L, _L)])
                return carry
            lax.fori_loop(0, _CH // _L, group_body, 0)

            pltpu.async_copy(rb, out_dst(g), osems[b])

        def outer(gp, carry):
            do_chunk(2 * gp, 0)
            do_chunk(2 * gp + 1, 1)
            return carry
        lax.fori_loop(0, n_chunks // 2, outer, 0)

        pltpu.make_async_copy(rb0, out_dst(n_chunks - 2), osems[0]).wait()
        pltpu.make_async_copy(rb1, out_dst(n_chunks - 1), osems[1]).wait()

    out = run(tokens, wf)
    return out.reshape(B, N, _D)
